# baseline (speedup 1.0000x reference)
import jax
import jax.numpy as jnp
from jax.experimental import pallas as pl
from jax.experimental.pallas import tpu as pltpu

_N, _D, _E, _S, _F = 2048, 768, 16, 128, 3072


def _moe_step(x_ref, se_ref, w1_ref, b1_ref, w2_ref, b2_ref, out_ref,
              pbuf_ref, ybuf_ref, rsum_ref):
    t = pl.program_id(0)
    x = x_ref[...]
    se = se_ref[0]

    logits = jax.lax.dot_general(
        x, se, (((1,), (1,)), ((), ())), preferred_element_type=jnp.float32)

    m = jnp.max(logits, axis=0, keepdims=True)
    p = jnp.exp(logits - m)
    pb = p.astype(jnp.bfloat16)
    colsum = jnp.sum(p, axis=0, keepdims=True)

    slot = t % 4
    pbuf_ref[:, pl.ds(slot * _S, _S)] = pb

    em_col = jnp.exp(m).reshape(_S, 1)
    csum = jnp.dot(p, em_col, preferred_element_type=jnp.float32)

    @pl.when(t == 0)
    def _():
        rsum_ref[...] = csum

    @pl.when(t > 0)
    def _():
        rsum_ref[...] += csum

    ps = jax.lax.dot_general(
        pb, x, (((0,), (0,)), ((), ())), preferred_element_type=jnp.float32)
    slots = ps * (1.0 / colsum).reshape(_S, 1)

    h = jax.nn.gelu(
        jnp.dot(slots, w1_ref[0], preferred_element_type=jnp.float32)
        + b1_ref[0])
    y = jnp.dot(h, w2_ref[0], preferred_element_type=jnp.float32) + b2_ref[0]
    ybuf_ref[pl.ds(slot * _S, _S), :] = (y * em_col).astype(jnp.bfloat16)

    @pl.when(t >= 2)
    def _():
        gd = t // 2 - 1
        base = (gd % 2) * (2 * _S)
        rows = pl.ds((t % 2) * (_N // 2), _N // 2)
        contrib = jnp.dot(pbuf_ref[rows, pl.ds(base, 2 * _S)],
                          ybuf_ref[pl.ds(base, 2 * _S), :],
                          preferred_element_type=jnp.float32)

        @pl.when(gd == 0)
        def _():
            out_ref[rows, :] = contrib

        @pl.when(gd > 0)
        def _():
            out_ref[rows, :] += contrib

    @pl.when(t == _E - 1)
    def _():
        base = ((_E // 2 - 1) % 2) * (2 * _S)
        out_ref[...] += jnp.dot(pbuf_ref[:, pl.ds(base, 2 * _S)],
                                ybuf_ref[pl.ds(base, 2 * _S), :],
                                preferred_element_type=jnp.float32)
        out_ref[...] = out_ref[...] * (1.0 / rsum_ref[...])


def kernel(x, slot_embeds, w1, b1, w2, b2):
    b, n, d = x.shape
    e, s, _ = slot_embeds.shape
    f = w1.shape[-1]
    x2 = x.reshape(n, d).astype(jnp.bfloat16)
    seb = slot_embeds.astype(jnp.bfloat16)
    b1r = b1.reshape(e, 1, f)
    b2r = b2.reshape(e, 1, d)

    out = pl.pallas_call(
        _moe_step,
        grid=(e,),
        in_specs=[
            pl.BlockSpec((n, d), lambda i: (0, 0)),
            pl.BlockSpec((1, s, d), lambda i: (i, 0, 0)),
            pl.BlockSpec((1, d, f), lambda i: (i, 0, 0)),
            pl.BlockSpec((1, 1, f), lambda i: (i, 0, 0)),
            pl.BlockSpec((1, f, d), lambda i: (i, 0, 0)),
            pl.BlockSpec((1, 1, d), lambda i: (i, 0, 0)),
        ],
        out_specs=pl.BlockSpec((n, d), lambda i: (0, 0)),
        out_shape=jax.ShapeDtypeStruct((n, d), jnp.float32),
        scratch_shapes=[
            pltpu.VMEM((n, 4 * s), jnp.bfloat16),
            pltpu.VMEM((4 * s, d), jnp.bfloat16),
            pltpu.VMEM((n, 1), jnp.float32),
        ],
        compiler_params=pltpu.CompilerParams(
            dimension_semantics=("arbitrary",)),
    )(x2, seb, w1, b1r, w2, b2r)
    return out.reshape(b, n, d)


# speedup vs baseline: 1.0517x; 1.0517x over previous
import jax
import jax.numpy as jnp
from jax.experimental import pallas as pl
from jax.experimental.pallas import tpu as pltpu

_N, _D, _E, _S, _F = 2048, 768, 16, 128, 3072


def _moe_step(x_ref, se_ref, w1_ref, b1_ref, w2_ref, b2_ref, out_ref,
              pbuf_ref, ybuf_ref, rsum_ref):
    t = pl.program_id(0)
    x = x_ref[...]
    se = se_ref[0].astype(jnp.bfloat16)

    logt = jax.lax.dot_general(
        se, x, (((1,), (1,)), ((), ())), preferred_element_type=jnp.float32)

    m = jnp.max(logt, axis=1, keepdims=True)
    pt = jnp.exp(logt - m)
    pbt = pt.astype(jnp.bfloat16)
    colsum = jnp.sum(pt, axis=1, keepdims=True)

    slot = t % 4
    pbuf_ref[pl.ds(slot * _S, _S), :] = pbt

    em_col = jnp.exp(m)
    csum = jax.lax.dot_general(
        em_col, pt, (((0,), (0,)), ((), ())),
        preferred_element_type=jnp.float32)

    @pl.when(t == 0)
    def _():
        rsum_ref[...] = csum

    @pl.when(t > 0)
    def _():
        rsum_ref[...] += csum

    ps = jax.lax.dot_general(
        pbt, x, (((1,), (0,)), ((), ())),
        preferred_element_type=jnp.float32)
    slots = ps * (1.0 / colsum)

    h = jax.nn.gelu(
        jnp.dot(slots, w1_ref[0], preferred_element_type=jnp.float32)
        + b1_ref[0])
    y = jnp.dot(h, w2_ref[0], preferred_element_type=jnp.float32) + b2_ref[0]
    ybuf_ref[pl.ds(slot * _S, _S), :] = (y * em_col).astype(jnp.bfloat16)

    @pl.when(t >= 2)
    def _():
        gd = t // 2 - 1
        base = (gd % 2) * (2 * _S)
        cols = pl.ds((t % 2) * (_N // 2), _N // 2)
        rows = pl.ds((t % 2) * (_N // 2), _N // 2)
        contrib = jax.lax.dot_general(
            pbuf_ref[pl.ds(base, 2 * _S), cols],
            ybuf_ref[pl.ds(base, 2 * _S), :],
            (((0,), (0,)), ((), ())),
            preferred_element_type=jnp.float32)

        @pl.when(gd == 0)
        def _():
            out_ref[rows, :] = contrib

        @pl.when(gd > 0)
        def _():
            out_ref[rows, :] += contrib

    @pl.when(t == _E - 1)
    def _():
        base = ((_E // 2 - 1) % 2) * (2 * _S)
        out_ref[...] += jax.lax.dot_general(
            pbuf_ref[pl.ds(base, 2 * _S), :],
            ybuf_ref[pl.ds(base, 2 * _S), :],
            (((0,), (0,)), ((), ())),
            preferred_element_type=jnp.float32)
        out_ref[...] = out_ref[...] * (1.0 / rsum_ref[...].reshape(_N, 1))


def kernel(x, slot_embeds, w1, b1, w2, b2):
    b, n, d = x.shape
    e, s, _ = slot_embeds.shape
    f = w1.shape[-1]
    x2 = x.reshape(n, d).astype(jnp.bfloat16)
    b1r = b1.reshape(e, 1, f)
    b2r = b2.reshape(e, 1, d)

    out = pl.pallas_call(
        _moe_step,
        grid=(e,),
        in_specs=[
            pl.BlockSpec((n, d), lambda i: (0, 0)),
            pl.BlockSpec((1, s, d), lambda i: (i, 0, 0)),
            pl.BlockSpec((1, d, f), lambda i: (i, 0, 0)),
            pl.BlockSpec((1, 1, f), lambda i: (i, 0, 0)),
            pl.BlockSpec((1, f, d), lambda i: (i, 0, 0)),
            pl.BlockSpec((1, 1, d), lambda i: (i, 0, 0)),
        ],
        out_specs=pl.BlockSpec((n, d), lambda i: (0, 0)),
        out_shape=jax.ShapeDtypeStruct((n, d), jnp.float32),
        scratch_shapes=[
            pltpu.VMEM((4 * s, n), jnp.bfloat16),
            pltpu.VMEM((4 * s, d), jnp.bfloat16),
            pltpu.VMEM((1, n), jnp.float32),
        ],
        compiler_params=pltpu.CompilerParams(
            dimension_semantics=("arbitrary",)),
    )(x2, slot_embeds, w1, b1r, w2, b2r)
    return out.reshape(b, n, d)
